# baseline (speedup 1.0000x reference)
import functools

import jax
import jax.numpy as jnp
from jax import lax
from jax.experimental import pallas as pl
from jax.experimental.pallas import tpu as pltpu
from jax.experimental.pallas import tpu_sc as plsc

N = 46080
E = 737280
B = 1024
PAD = 45
OUT = 128
WC = 24

NC = 2
NS = 16
BS = 256
K = 4
EPT = E // NS
NR = EPT // (K * BS)
ROWS_PT = N // NS
ZR = 320

ROWS_BLK = 2304


def _make_seg_sum(n_chunks):
    cpc = n_chunks // NC
    mesh = plsc.VectorSubcoreMesh(core_axis_name="c", subcore_axis_name="s")

    @functools.partial(
        pl.kernel,
        mesh=mesh,
        out_type=jax.ShapeDtypeStruct((n_chunks * N, WC), jnp.float32),
        scratch_types=[
            pltpu.VMEM((K * BS,), jnp.int32),
            [pltpu.VMEM((BS,), jnp.int32) for _ in range(K)],
            [pltpu.VMEM((BS, WC), jnp.float32) for _ in range(K)],
            pltpu.VMEM((ZR, WC), jnp.float32),
            pltpu.VMEM_SHARED((N, WC), jnp.float32),
            pltpu.SemaphoreType.DMA,
            [pltpu.SemaphoreType.DMA for _ in range(K)],
            [pltpu.SemaphoreType.DMA for _ in range(K)],
            [pltpu.SemaphoreType.DMA for _ in range(K)],
        ],
        compiler_params=pltpu.CompilerParams(use_tc_tiling_on_sc=False),
    )
    def seg(table, srcr, dstr, zsrc, out, idx_s, idx_d, rows, zbuf, acc,
            isem, dsem, gsem, ssem):
        c = lax.axis_index("c")
        s = lax.axis_index("s")
        tile_base = s * EPT
        pltpu.sync_copy(zsrc, zbuf)
        for k in range(n_chunks):

            @pl.when(k // cpc == c)
            def _():
                for z in range(ROWS_PT // ZR):
                    pltpu.sync_copy(
                        zbuf, acc.at[pl.ds(s * ROWS_PT + z * ZR, ZR)])
                plsc.subcore_barrier()
                off = k * N

                def round_body(g, _):
                    @pl.when(g > 0)
                    def _():
                        for j in range(K):
                            pltpu.make_async_copy(
                                rows[j], acc.at[idx_d[j]], ssem[j]).wait()

                    base = tile_base + g * (K * BS)
                    ih = pltpu.async_copy(
                        srcr.at[pl.ds(base, K * BS)], idx_s, isem)
                    dh = [
                        pltpu.async_copy(
                            dstr.at[pl.ds(base + j * BS, BS)], idx_d[j],
                            dsem[j])
                        for j in range(K)
                    ]
                    ih.wait()
                    if off:
                        def addoff(j2, _):
                            sl = pl.ds(j2 * 16, 16)
                            idx_s[sl] = idx_s[sl] + off
                            return 0

                        lax.fori_loop(0, K * BS // 16, addoff, 0, unroll=8)
                    gh = [
                        pltpu.async_copy(
                            table.at[idx_s.at[pl.ds(j * BS, BS)]],
                            rows[j], gsem[j])
                        for j in range(K)
                    ]
                    for j in range(K):
                        gh[j].wait()
                        dh[j].wait()
                        pltpu.async_copy(rows[j], acc.at[idx_d[j]],
                                         ssem[j], add=True)
                    return 0

                lax.fori_loop(0, NR, round_body, 0)
                for j in range(K):
                    pltpu.make_async_copy(
                        rows[j], acc.at[idx_d[j]], ssem[j]).wait()
                plsc.subcore_barrier()
                pltpu.sync_copy(
                    acc.at[pl.ds(s * ROWS_PT, ROWS_PT)],
                    out.at[pl.ds(k * N + s * ROWS_PT, ROWS_PT)],
                )

    return seg


def _mlp_body(ci, co, last, h_ref, agg_ref, rw_ref, rb_ref, w1_ref, b1_ref,
              w2_ref, b2_ref, out_ref):
    h = jnp.concatenate([h_ref[cc] for cc in range(ci)], axis=-1)
    agg = jnp.concatenate([agg_ref[cc] for cc in range(ci)], axis=-1)
    res = jnp.dot(h, rw_ref[...], preferred_element_type=jnp.float32)
    t = jnp.dot(h + agg, w1_ref[...], preferred_element_type=jnp.float32)
    t = jnp.maximum(t + b1_ref[...], 0.0)
    t = jnp.dot(t, w2_ref[...], preferred_element_type=jnp.float32)
    y = res + rb_ref[...] + t + b2_ref[...]
    if last:
        out_ref[...] = y
    else:
        for cc in range(co):
            out_ref[cc] = y[:, cc * WC:(cc + 1) * WC]


def _layer_mlp(hc3, aggc3, rw, rb, w1, b1, w2, b2, ci, co, last):
    wo = rw.shape[1]
    grid = (N // ROWS_BLK,)
    chunk_spec = pl.BlockSpec((ci, ROWS_BLK, WC), lambda i: (0, i, 0))
    full = lambda a: pl.BlockSpec(a.shape, lambda i: (0,) * a.ndim)
    if last:
        out_specs = pl.BlockSpec((ROWS_BLK, wo), lambda i: (i, 0))
        out_shape = jax.ShapeDtypeStruct((N, wo), jnp.float32)
    else:
        out_specs = pl.BlockSpec((co, ROWS_BLK, WC), lambda i: (0, i, 0))
        out_shape = jax.ShapeDtypeStruct((co, N, WC), jnp.float32)
    return pl.pallas_call(
        functools.partial(_mlp_body, ci, co, last),
        grid=grid,
        in_specs=[
            chunk_spec,
            chunk_spec,
            full(rw),
            full(rb),
            full(w1),
            full(b1),
            full(w2),
            full(b2),
        ],
        out_specs=out_specs,
        out_shape=out_shape,
    )(hc3, aggc3, rw, rb, w1, b1, w2, b2)


def _pad2(a, r, c):
    return jnp.pad(a, ((0, r - a.shape[0]), (0, c - a.shape[1])))


def kernel(x, edge_index, ptr, res_w1, res_b1, nn_w1a, nn_b1a, nn_w1b, nn_b1b,
           res_w2, res_b2, nn_w2a, nn_b2a, nn_w2b, nn_b2b, res_w3, res_b3,
           nn_w3a, nn_b3a, nn_w3b, nn_b3b, res_w4, res_b4, nn_w4a, nn_b4a,
           nn_w4b, nn_b4b):
    src = edge_index[0]
    dst = edge_index[1]

    params = [
        (res_w1, res_b1, nn_w1a, nn_b1a, nn_w1b, nn_b1b),
        (res_w2, res_b2, nn_w2a, nn_b2a, nn_w2b, nn_b2b),
        (res_w3, res_b3, nn_w3a, nn_b3a, nn_w3b, nn_b3b),
        (res_w4, res_b4, nn_w4a, nn_b4a, nn_w4b, nn_b4b),
    ]
    widths = [96, 144, 144, 144, 128]
    chunks = [4, 6, 6, 6]

    h0 = _pad2(x, N, widths[0])
    hc3 = h0.reshape(N, chunks[0], WC).transpose(1, 0, 2)

    for l, (rw, rb, w1, b1, w2, b2) in enumerate(params):
        wi, wo = widths[l], widths[l + 1]
        ci = chunks[l]
        co = chunks[l + 1] if l < 3 else 0
        last = l == 3

        rwp = _pad2(rw, wi, wo)
        w1p = _pad2(w1, wi, wo)
        w2p = _pad2(w2, wo, wo)
        rbp = jnp.pad(rb, (0, wo - rb.shape[0])).reshape(1, wo)
        b1p = jnp.pad(b1, (0, wo - b1.shape[0])).reshape(1, wo)
        b2p = jnp.pad(b2, (0, wo - b2.shape[0])).reshape(1, wo)

        table = hc3.reshape(ci * N, WC)
        agg2 = _make_seg_sum(ci)(table, src, dst,
                                 jnp.zeros((ZR, WC), jnp.float32))
        aggc3 = agg2.reshape(ci, N, WC)
        nxt = _layer_mlp(hc3, aggc3, rwp, rbp, w1p, b1p, w2p, b2p, ci, co,
                         last)
        if last:
            h4 = nxt
        else:
            hc3 = nxt

    return h4.reshape(B, PAD, OUT)


# speedup vs baseline: 1.0398x; 1.0398x over previous
import functools

import jax
import jax.numpy as jnp
from jax import lax
from jax.experimental import pallas as pl
from jax.experimental.pallas import tpu as pltpu
from jax.experimental.pallas import tpu_sc as plsc

N = 46080
E = 737280
B = 1024
PAD = 45
OUT = 128
WC = 24

NC = 2
NS = 16
BS = 128
K = 8
EPT = E // NS
NR = EPT // (K * BS)
ROWS_PT = N // NS
ZR = 320

ROWS_BLK = 2304


def _make_seg_sum(n_chunks):
    cpc = n_chunks // NC
    mesh = plsc.VectorSubcoreMesh(core_axis_name="c", subcore_axis_name="s")

    @functools.partial(
        pl.kernel,
        mesh=mesh,
        out_type=jax.ShapeDtypeStruct((n_chunks * N, WC), jnp.float32),
        scratch_types=[
            pltpu.VMEM((K * BS,), jnp.int32),
            [pltpu.VMEM((BS,), jnp.int32) for _ in range(K)],
            [pltpu.VMEM((BS, WC), jnp.float32) for _ in range(K)],
            pltpu.VMEM((ZR, WC), jnp.float32),
            pltpu.VMEM_SHARED((N, WC), jnp.float32),
            pltpu.SemaphoreType.DMA,
            [pltpu.SemaphoreType.DMA for _ in range(K)],
            [pltpu.SemaphoreType.DMA for _ in range(K)],
            [pltpu.SemaphoreType.DMA for _ in range(K)],
        ],
        compiler_params=pltpu.CompilerParams(use_tc_tiling_on_sc=False),
    )
    def seg(table, srcr, dstr, zsrc, out, idx_s, idx_d, rows, zbuf, acc,
            isem, dsem, gsem, ssem):
        c = lax.axis_index("c")
        s = lax.axis_index("s")
        tile_base = s * EPT
        pltpu.sync_copy(zsrc, zbuf)
        for k in range(n_chunks):

            @pl.when(k // cpc == c)
            def _():
                for z in range(ROWS_PT // ZR):
                    pltpu.sync_copy(
                        zbuf, acc.at[pl.ds(s * ROWS_PT + z * ZR, ZR)])
                plsc.subcore_barrier()
                off = k * N

                def round_body(g, _):
                    @pl.when(g > 0)
                    def _():
                        for j in range(K):
                            pltpu.make_async_copy(
                                rows[j], acc.at[idx_d[j]], ssem[j]).wait()

                    base = tile_base + g * (K * BS)
                    ih = pltpu.async_copy(
                        srcr.at[pl.ds(base, K * BS)], idx_s, isem)
                    dh = [
                        pltpu.async_copy(
                            dstr.at[pl.ds(base + j * BS, BS)], idx_d[j],
                            dsem[j])
                        for j in range(K)
                    ]
                    ih.wait()
                    if off:
                        def addoff(j2, _):
                            sl = pl.ds(j2 * 16, 16)
                            idx_s[sl] = idx_s[sl] + off
                            return 0

                        lax.fori_loop(0, K * BS // 16, addoff, 0, unroll=8)
                    gh = [
                        pltpu.async_copy(
                            table.at[idx_s.at[pl.ds(j * BS, BS)]],
                            rows[j], gsem[j])
                        for j in range(K)
                    ]
                    for j in range(K):
                        gh[j].wait()
                        dh[j].wait()
                        pltpu.async_copy(rows[j], acc.at[idx_d[j]],
                                         ssem[j], add=True)
                    return 0

                lax.fori_loop(0, NR, round_body, 0)
                for j in range(K):
                    pltpu.make_async_copy(
                        rows[j], acc.at[idx_d[j]], ssem[j]).wait()
                plsc.subcore_barrier()
                pltpu.sync_copy(
                    acc.at[pl.ds(s * ROWS_PT, ROWS_PT)],
                    out.at[pl.ds(k * N + s * ROWS_PT, ROWS_PT)],
                )

    return seg


def _mlp_body(ci, co, last, h_ref, agg_ref, rw_ref, rb_ref, w1_ref, b1_ref,
              w2_ref, b2_ref, out_ref):
    h = jnp.concatenate([h_ref[cc] for cc in range(ci)], axis=-1)
    agg = jnp.concatenate([agg_ref[cc] for cc in range(ci)], axis=-1)
    res = jnp.dot(h, rw_ref[...], preferred_element_type=jnp.float32)
    t = jnp.dot(h + agg, w1_ref[...], preferred_element_type=jnp.float32)
    t = jnp.maximum(t + b1_ref[...], 0.0)
    t = jnp.dot(t, w2_ref[...], preferred_element_type=jnp.float32)
    y = res + rb_ref[...] + t + b2_ref[...]
    if last:
        out_ref[...] = y
    else:
        for cc in range(co):
            out_ref[cc] = y[:, cc * WC:(cc + 1) * WC]


def _layer_mlp(hc3, aggc3, rw, rb, w1, b1, w2, b2, ci, co, last):
    wo = rw.shape[1]
    grid = (N // ROWS_BLK,)
    chunk_spec = pl.BlockSpec((ci, ROWS_BLK, WC), lambda i: (0, i, 0))
    full = lambda a: pl.BlockSpec(a.shape, lambda i: (0,) * a.ndim)
    if last:
        out_specs = pl.BlockSpec((ROWS_BLK, wo), lambda i: (i, 0))
        out_shape = jax.ShapeDtypeStruct((N, wo), jnp.float32)
    else:
        out_specs = pl.BlockSpec((co, ROWS_BLK, WC), lambda i: (0, i, 0))
        out_shape = jax.ShapeDtypeStruct((co, N, WC), jnp.float32)
    return pl.pallas_call(
        functools.partial(_mlp_body, ci, co, last),
        grid=grid,
        in_specs=[
            chunk_spec,
            chunk_spec,
            full(rw),
            full(rb),
            full(w1),
            full(b1),
            full(w2),
            full(b2),
        ],
        out_specs=out_specs,
        out_shape=out_shape,
    )(hc3, aggc3, rw, rb, w1, b1, w2, b2)


def _pad2(a, r, c):
    return jnp.pad(a, ((0, r - a.shape[0]), (0, c - a.shape[1])))


def kernel(x, edge_index, ptr, res_w1, res_b1, nn_w1a, nn_b1a, nn_w1b, nn_b1b,
           res_w2, res_b2, nn_w2a, nn_b2a, nn_w2b, nn_b2b, res_w3, res_b3,
           nn_w3a, nn_b3a, nn_w3b, nn_b3b, res_w4, res_b4, nn_w4a, nn_b4a,
           nn_w4b, nn_b4b):
    src = edge_index[0]
    dst = edge_index[1]

    params = [
        (res_w1, res_b1, nn_w1a, nn_b1a, nn_w1b, nn_b1b),
        (res_w2, res_b2, nn_w2a, nn_b2a, nn_w2b, nn_b2b),
        (res_w3, res_b3, nn_w3a, nn_b3a, nn_w3b, nn_b3b),
        (res_w4, res_b4, nn_w4a, nn_b4a, nn_w4b, nn_b4b),
    ]
    widths = [96, 144, 144, 144, 128]
    chunks = [4, 6, 6, 6]

    h0 = _pad2(x, N, widths[0])
    hc3 = h0.reshape(N, chunks[0], WC).transpose(1, 0, 2)

    for l, (rw, rb, w1, b1, w2, b2) in enumerate(params):
        wi, wo = widths[l], widths[l + 1]
        ci = chunks[l]
        co = chunks[l + 1] if l < 3 else 0
        last = l == 3

        rwp = _pad2(rw, wi, wo)
        w1p = _pad2(w1, wi, wo)
        w2p = _pad2(w2, wo, wo)
        rbp = jnp.pad(rb, (0, wo - rb.shape[0])).reshape(1, wo)
        b1p = jnp.pad(b1, (0, wo - b1.shape[0])).reshape(1, wo)
        b2p = jnp.pad(b2, (0, wo - b2.shape[0])).reshape(1, wo)

        table = hc3.reshape(ci * N, WC)
        agg2 = _make_seg_sum(ci)(table, src, dst,
                                 jnp.zeros((ZR, WC), jnp.float32))
        aggc3 = agg2.reshape(ci, N, WC)
        nxt = _layer_mlp(hc3, aggc3, rwp, rbp, w1p, b1p, w2p, b2p, ci, co,
                         last)
        if last:
            h4 = nxt
        else:
            hc3 = nxt

    return h4.reshape(B, PAD, OUT)
